# Initial kernel scaffold; baseline (speedup 1.0000x reference)
import functools

import jax
import jax.numpy as jnp
from jax import lax
from jax.experimental import pallas as pl
from jax.experimental.pallas import tpu as pltpu
from jax.experimental.pallas import tpu_sc as plsc

NUM_NODES = 10000
NUM_EDGES = 320000
D = 128

NC = 2
NS = 16
NW = NC * NS

BLK = 128
BLKS_PER_CHUNK = 4
CHUNK = BLK * BLKS_PER_CHUNK
NUM_CHUNKS = NUM_EDGES // CHUNK
ITERS = (NUM_CHUNKS + NW - 1) // NW
ROWS_PER_TILE = NUM_NODES // NS


def _sc_partials(E2d, ids2d, zeros2d):
    mesh = plsc.VectorSubcoreMesh(core_axis_name="c", subcore_axis_name="s")

    @functools.partial(
        pl.kernel,
        out_type=jax.ShapeDtypeStruct((NC, NUM_NODES, D), jnp.float32),
        mesh=mesh,
        scratch_types=[
            pltpu.VMEM((CHUNK, D), jnp.float32),
            pltpu.VMEM((BLKS_PER_CHUNK, BLK), jnp.int32),
            pltpu.VMEM_SHARED((NUM_NODES, D), jnp.float32),
        ],
    )
    def k(e_hbm, ids_hbm, zeros_hbm, out_hbm, rows_v, idx_v, acc_s):
        cid = lax.axis_index("c")
        sid = lax.axis_index("s")
        wid = sid * NC + cid

        pltpu.sync_copy(
            zeros_hbm.at[pl.ds(sid * ROWS_PER_TILE, ROWS_PER_TILE)],
            acc_s.at[pl.ds(sid * ROWS_PER_TILE, ROWS_PER_TILE)],
        )
        plsc.subcore_barrier()

        def body(i, _):
            j = i * NW + wid

            @pl.when(j < NUM_CHUNKS)
            def _():
                pltpu.sync_copy(ids_hbm.at[pl.ds(j * BLKS_PER_CHUNK, BLKS_PER_CHUNK)], idx_v)
                pltpu.sync_copy(e_hbm.at[pl.ds(j * CHUNK, CHUNK)], rows_v)
                for b in range(BLKS_PER_CHUNK):
                    pltpu.sync_copy(
                        rows_v.at[pl.ds(b * BLK, BLK)],
                        acc_s.at[idx_v.at[b]],
                        add=True,
                    )

            return ()

        lax.fori_loop(0, ITERS, body, ())
        plsc.subcore_barrier()

        pltpu.sync_copy(
            acc_s.at[pl.ds(sid * ROWS_PER_TILE, ROWS_PER_TILE)],
            out_hbm.at[cid, pl.ds(sid * ROWS_PER_TILE, ROWS_PER_TILE)],
        )

    return k(E2d, ids2d, zeros2d)


def _combine_body(p_ref, o_ref):
    o_ref[...] = p_ref[0] + p_ref[1]


def _combine(partials):
    blk = 1000
    return pl.pallas_call(
        _combine_body,
        grid=(NUM_NODES // blk,),
        in_specs=[pl.BlockSpec((NC, blk, D), lambda i: (0, i, 0))],
        out_specs=pl.BlockSpec((blk, D), lambda i: (i, 0)),
        out_shape=jax.ShapeDtypeStruct((NUM_NODES, D), jnp.float32),
    )(partials)


@jax.jit
def kernel(V_set, E_set, node_ids):
    E2d = E_set[0]
    ids2d = node_ids[0].reshape(NUM_EDGES // BLK, BLK)
    zeros2d = jnp.zeros((NUM_NODES, D), jnp.float32)
    partials = _sc_partials(E2d, ids2d, zeros2d)
    out = _combine(partials)
    return out[jnp.newaxis]


# speedup vs baseline: 5.4183x; 5.4183x over previous
import functools

import jax
import jax.numpy as jnp
from jax import lax
from jax.experimental import pallas as pl
from jax.experimental.pallas import tpu as pltpu
from jax.experimental.pallas import tpu_sc as plsc

NUM_NODES = 10000
NUM_EDGES = 320000
D = 128

NC = 2
NS = 16
NW = NC * NS

BLK = 128
BLKS_PER_CHUNK = 2
CHUNK = BLK * BLKS_PER_CHUNK
NUM_CHUNKS = NUM_EDGES // CHUNK
ITERS = (NUM_CHUNKS + NW - 1) // NW
OUT_ROWS = 640
OUT_ROWS_LAST = NUM_NODES - OUT_ROWS * (NS - 1)


def _sc_partials(E2d, ids2d, zeros2d):
    mesh = plsc.VectorSubcoreMesh(core_axis_name="c", subcore_axis_name="s")

    @functools.partial(
        pl.kernel,
        out_type=jax.ShapeDtypeStruct((NC, NUM_NODES, D), jnp.float32),
        mesh=mesh,
        scratch_types=[
            pltpu.VMEM((CHUNK, D), jnp.float32),
            pltpu.VMEM((BLKS_PER_CHUNK, BLK), jnp.int32),
            pltpu.VMEM_SHARED((NUM_NODES, D), jnp.float32),
        ],
    )
    def k(e_hbm, ids_hbm, zeros_hbm, out_hbm, rows_v, idx_v, acc_s):
        cid = lax.axis_index("c")
        sid = lax.axis_index("s")
        wid = sid * NC + cid

        @pl.when(sid < NS - 1)
        def _():
            pltpu.sync_copy(
                zeros_hbm.at[pl.ds(sid * OUT_ROWS, OUT_ROWS)],
                acc_s.at[pl.ds(sid * OUT_ROWS, OUT_ROWS)],
            )

        @pl.when(sid == NS - 1)
        def _():
            pltpu.sync_copy(
                zeros_hbm.at[pl.ds((NS - 1) * OUT_ROWS, OUT_ROWS_LAST)],
                acc_s.at[pl.ds((NS - 1) * OUT_ROWS, OUT_ROWS_LAST)],
            )

        plsc.subcore_barrier()

        def body(i, _):
            j = i * NW + wid

            @pl.when(j < NUM_CHUNKS)
            def _():
                pltpu.sync_copy(ids_hbm.at[j], idx_v)
                pltpu.sync_copy(e_hbm.at[pl.ds(j * CHUNK, CHUNK)], rows_v)
                for b in range(BLKS_PER_CHUNK):
                    pltpu.sync_copy(
                        rows_v.at[pl.ds(b * BLK, BLK)],
                        acc_s.at[idx_v.at[b]],
                        add=True,
                    )

            return ()

        lax.fori_loop(0, ITERS, body, ())
        plsc.subcore_barrier()

        @pl.when(sid < NS - 1)
        def _():
            pltpu.sync_copy(
                acc_s.at[pl.ds(sid * OUT_ROWS, OUT_ROWS)],
                out_hbm.at[cid, pl.ds(sid * OUT_ROWS, OUT_ROWS)],
            )

        @pl.when(sid == NS - 1)
        def _():
            pltpu.sync_copy(
                acc_s.at[pl.ds((NS - 1) * OUT_ROWS, OUT_ROWS_LAST)],
                out_hbm.at[cid, pl.ds((NS - 1) * OUT_ROWS, OUT_ROWS_LAST)],
            )

    return k(E2d, ids2d, zeros2d)


def _combine_body(p_ref, o_ref):
    o_ref[...] = p_ref[0] + p_ref[1]


def _combine(partials):
    blk = 1000
    return pl.pallas_call(
        _combine_body,
        grid=(NUM_NODES // blk,),
        in_specs=[pl.BlockSpec((NC, blk, D), lambda i: (0, i, 0))],
        out_specs=pl.BlockSpec((blk, D), lambda i: (i, 0)),
        out_shape=jax.ShapeDtypeStruct((NUM_NODES, D), jnp.float32),
    )(partials)


@jax.jit
def kernel(V_set, E_set, node_ids):
    E2d = E_set[0]
    ids2d = node_ids[0].reshape(NUM_CHUNKS, BLKS_PER_CHUNK, BLK)
    zeros2d = jnp.zeros((NUM_NODES, D), jnp.float32)
    partials = _sc_partials(E2d, ids2d, zeros2d)
    out = _combine(partials)
    return out[jnp.newaxis]


# speedup vs baseline: 7.8466x; 1.4482x over previous
import functools

import jax
import jax.numpy as jnp
from jax import lax
from jax.experimental import pallas as pl
from jax.experimental.pallas import tpu as pltpu
from jax.experimental.pallas import tpu_sc as plsc

NUM_NODES = 10000
NUM_EDGES = 320000
D = 128

NC = 2
NS = 16
NW = NC * NS

BLK = 128
NUM_CHUNKS = NUM_EDGES // BLK
ITERS = (NUM_CHUNKS + NW - 1) // NW
NBUF = 2
OUT_ROWS = 640
OUT_ROWS_LAST = NUM_NODES - OUT_ROWS * (NS - 1)


def _sc_partials(E2d, ids2d, zeros2d):
    mesh = plsc.VectorSubcoreMesh(core_axis_name="c", subcore_axis_name="s")

    @functools.partial(
        pl.kernel,
        out_type=jax.ShapeDtypeStruct((NC, NUM_NODES, D), jnp.float32),
        mesh=mesh,
        scratch_types=[
            pltpu.VMEM((NBUF, BLK, D), jnp.float32),
            pltpu.VMEM((NBUF, 1, BLK), jnp.int32),
            pltpu.VMEM_SHARED((NUM_NODES, D), jnp.float32),
            pltpu.SemaphoreType.DMA((NBUF,)),
            pltpu.SemaphoreType.DMA((NBUF,)),
        ],
    )
    def k(e_hbm, ids_hbm, zeros_hbm, out_hbm, rows_v, idx_v, acc_s, sem_r, sem_i):
        cid = lax.axis_index("c")
        sid = lax.axis_index("s")
        wid = sid * NC + cid

        @pl.when(sid < NS - 1)
        def _():
            pltpu.sync_copy(
                zeros_hbm.at[pl.ds(sid * OUT_ROWS, OUT_ROWS)],
                acc_s.at[pl.ds(sid * OUT_ROWS, OUT_ROWS)],
            )

        @pl.when(sid == NS - 1)
        def _():
            pltpu.sync_copy(
                zeros_hbm.at[pl.ds((NS - 1) * OUT_ROWS, OUT_ROWS_LAST)],
                acc_s.at[pl.ds((NS - 1) * OUT_ROWS, OUT_ROWS_LAST)],
            )

        plsc.subcore_barrier()

        def start(it, b):
            j = it * NW + wid

            @pl.when(j < NUM_CHUNKS)
            def _():
                pltpu.async_copy(ids_hbm.at[j], idx_v.at[b], sem_i.at[b])
                pltpu.async_copy(
                    e_hbm.at[pl.ds(j * BLK, BLK)], rows_v.at[b], sem_r.at[b]
                )

        def finish(it, b):
            j = it * NW + wid

            @pl.when(j < NUM_CHUNKS)
            def _():
                pltpu.make_async_copy(ids_hbm.at[j], idx_v.at[b], sem_i.at[b]).wait()
                pltpu.make_async_copy(
                    e_hbm.at[pl.ds(j * BLK, BLK)], rows_v.at[b], sem_r.at[b]
                ).wait()
                pltpu.sync_copy(rows_v.at[b], acc_s.at[idx_v.at[b, 0]], add=True)

        start(0, 0)

        def body(k, _):
            for b in range(NBUF):
                it = k * NBUF + b
                start(it + 1, (b + 1) % NBUF)
                finish(it, b)
            return ()

        lax.fori_loop(0, (ITERS + NBUF - 1) // NBUF, body, ())
        plsc.subcore_barrier()

        @pl.when(sid < NS - 1)
        def _():
            pltpu.sync_copy(
                acc_s.at[pl.ds(sid * OUT_ROWS, OUT_ROWS)],
                out_hbm.at[cid, pl.ds(sid * OUT_ROWS, OUT_ROWS)],
            )

        @pl.when(sid == NS - 1)
        def _():
            pltpu.sync_copy(
                acc_s.at[pl.ds((NS - 1) * OUT_ROWS, OUT_ROWS_LAST)],
                out_hbm.at[cid, pl.ds((NS - 1) * OUT_ROWS, OUT_ROWS_LAST)],
            )

    return k(E2d, ids2d, zeros2d)


def _combine_body(p_ref, o_ref):
    o_ref[...] = p_ref[0] + p_ref[1]


def _combine(partials):
    blk = 1000
    return pl.pallas_call(
        _combine_body,
        grid=(NUM_NODES // blk,),
        in_specs=[pl.BlockSpec((NC, blk, D), lambda i: (0, i, 0))],
        out_specs=pl.BlockSpec((blk, D), lambda i: (i, 0)),
        out_shape=jax.ShapeDtypeStruct((NUM_NODES, D), jnp.float32),
    )(partials)


@jax.jit
def kernel(V_set, E_set, node_ids):
    E2d = E_set[0]
    ids2d = node_ids[0].reshape(NUM_CHUNKS, 1, BLK)
    zeros2d = jnp.zeros((NUM_NODES, D), jnp.float32)
    partials = _sc_partials(E2d, ids2d, zeros2d)
    out = _combine(partials)
    return out[jnp.newaxis]
